# baseline (speedup 1.0000x reference)
import jax
import jax.numpy as jnp
from jax import lax
from jax.experimental import pallas as pl
from jax.experimental.pallas import tpu as pltpu
from jax.experimental.pallas import tpu_sc as plsc

N_ROWS = 10000
D = 128
NUM_GRAPHS = 64
SEGS = 72
NC, NS = 2, 16
NW = NC * NS
N_PAD = 10240
ROWS_PER = N_PAD // NW
CH = 80
NCH = ROWS_PER // CH
GPC = CH // 16


def _sc_body(x_hbm, b_hbm, z_hbm, ps_hbm, pc_hbm,
             x_v, ids_v, cnt_v, acc_sh, sem, xsem):
    cid = lax.axis_index("c")
    sid = lax.axis_index("s")
    wid = sid * NC + cid
    base = wid * ROWS_PER

    @pl.when(sid == 0)
    def _():
        pltpu.sync_copy(z_hbm, acc_sh)

    for j in range(NCH):
        pltpu.sync_copy(b_hbm.at[pl.ds(base + j * CH, CH)], ids_v.at[j])
    pltpu.async_copy(x_hbm.at[pl.ds(base, ROWS_PER)], x_v, xsem)

    zero16 = jnp.zeros((16,), jnp.float32)
    ones16 = jnp.ones((16,), jnp.float32)

    def zb(i, c):
        cnt_v[i] = zero16
        return c

    lax.fori_loop(0, SEGS, zb, 0)

    def grp(g, c):
        idv = ids_v[g // GPC, pl.ds((g % GPC) * 16, 16)]
        for l in range(16):
            plsc.addupdate(cnt_v.at[idv[l]], ones16)
        return c

    lax.fori_loop(0, ROWS_PER // 16, grp, 0)

    pltpu.make_async_copy(x_hbm.at[pl.ds(base, ROWS_PER)], x_v, xsem).wait()
    plsc.subcore_barrier()

    for j in range(NCH):
        pltpu.async_copy(x_v.at[pl.ds(j * CH, CH)],
                         acc_sh.at[ids_v.at[j]], sem, add=True)
    for j in range(NCH):
        pltpu.make_async_copy(x_v.at[pl.ds(j * CH, CH)],
                              acc_sh.at[ids_v.at[j]], sem).wait()

    plsc.subcore_barrier()

    @pl.when(sid == 0)
    def _():
        pltpu.sync_copy(acc_sh.at[pl.ds(0, NUM_GRAPHS)], ps_hbm.at[cid])

    pltpu.sync_copy(cnt_v.at[pl.ds(0, NUM_GRAPHS)], pc_hbm.at[wid])


@jax.jit
def _segment_partials(x_p, b_p, z):
    mesh = plsc.VectorSubcoreMesh(core_axis_name="c", subcore_axis_name="s",
                                  num_cores=NC, num_subcores=NS)
    f = pl.kernel(
        _sc_body,
        out_type=(
            jax.ShapeDtypeStruct((NC, NUM_GRAPHS, D), jnp.float32),
            jax.ShapeDtypeStruct((NW, NUM_GRAPHS, 16), jnp.float32),
        ),
        mesh=mesh,
        scratch_types=[
            pltpu.VMEM((ROWS_PER, D), jnp.float32),
            pltpu.VMEM((NCH, CH), jnp.int32),
            pltpu.VMEM((SEGS, 16), jnp.float32),
            pltpu.VMEM_SHARED((SEGS, D), jnp.float32),
            pltpu.SemaphoreType.DMA,
            pltpu.SemaphoreType.DMA,
        ],
    )
    return f(x_p, b_p, z)


def _tc_body(ps_ref, pc_ref, u_ref, w1_ref, b1_ref, w2_ref, b2_ref, y_ref):
    sums = ps_ref[0] + ps_ref[1]
    cnt = jnp.sum(pc_ref[...], axis=0)[:, :1]
    agg = sums / jnp.maximum(cnt, 1.0)
    out = jnp.concatenate([u_ref[...], agg], axis=1)
    h = jnp.maximum(out @ w1_ref[...] + b1_ref[...], 0.0)
    y_ref[...] = h @ w2_ref[...] + b2_ref[...]


@jax.jit
def _pool_mlp(ps, pc, u, W1, b1, W2, b2):
    return pl.pallas_call(
        _tc_body,
        out_shape=jax.ShapeDtypeStruct((NUM_GRAPHS, 32), jnp.float32),
    )(ps, pc, u, W1, b1.reshape(1, 32), W2, b2.reshape(1, 32))


def kernel(x, edge_index, edge_attr, u, batch, W1, b1, W2, b2):
    del edge_index, edge_attr
    x_p = jnp.pad(x, ((0, N_PAD - N_ROWS), (0, 0)))
    b_p = jnp.pad(batch.astype(jnp.int32), (0, N_PAD - N_ROWS),
                  constant_values=NUM_GRAPHS)
    z = jnp.zeros((SEGS, D), jnp.float32)
    ps, pc = _segment_partials(x_p, b_p, z)
    return _pool_mlp(ps, pc, u, W1, b1, W2, b2)


# speedup vs baseline: 1.0871x; 1.0871x over previous
import jax
import jax.numpy as jnp
from jax import lax
from jax.experimental import pallas as pl
from jax.experimental.pallas import tpu as pltpu
from jax.experimental.pallas import tpu_sc as plsc

N_ROWS = 10000
D = 128
NUM_GRAPHS = 64
NC, NS = 2, 16
NW = NC * NS
ROWS_PER = 320
TAIL = N_ROWS - ROWS_PER * (NW - 1)
CH = 80
NCH = ROWS_PER // CH


def _sc_body(x_hbm, b_hbm, z128_hbm, ps_hbm, pc_hbm,
             x_v, ids_v, cnt_v, acc_sh, sem, xsem):
    cid = lax.axis_index("c")
    sid = lax.axis_index("s")
    wid = sid * NC + cid
    base = wid * ROWS_PER
    is_tail = wid == NW - 1

    @pl.when(sid == 0)
    def _():
        pltpu.sync_copy(z128_hbm, acc_sh)

    @pl.when(jnp.logical_not(is_tail))
    def _():
        for j in range(NCH):
            pltpu.sync_copy(b_hbm.at[pl.ds(base + j * CH, CH)], ids_v.at[j])
        pltpu.async_copy(x_hbm.at[pl.ds(base, ROWS_PER)], x_v, xsem)

    @pl.when(is_tail)
    def _():
        pltpu.sync_copy(b_hbm.at[pl.ds(base, TAIL)], ids_v.at[0])
        pltpu.async_copy(x_hbm.at[pl.ds(base, TAIL)],
                         x_v.at[pl.ds(0, TAIL)], xsem)

    zero16 = jnp.zeros((16,), jnp.float32)
    ones16 = jnp.ones((16,), jnp.float32)

    def zb(i, c):
        cnt_v[i] = zero16
        return c

    lax.fori_loop(0, NUM_GRAPHS, zb, 0)

    def grp(g, c):
        idv = ids_v[g // (CH // 16), pl.ds((g % (CH // 16)) * 16, 16)]
        for l in range(16):
            plsc.addupdate(cnt_v.at[idv[l]], ones16)
        return c

    ngrp = jnp.where(is_tail, TAIL // 16, ROWS_PER // 16)
    lax.fori_loop(0, ngrp, grp, 0)

    @pl.when(jnp.logical_not(is_tail))
    def _():
        pltpu.make_async_copy(x_hbm.at[pl.ds(base, ROWS_PER)], x_v, xsem).wait()

    @pl.when(is_tail)
    def _():
        pltpu.make_async_copy(x_hbm.at[pl.ds(base, TAIL)],
                              x_v.at[pl.ds(0, TAIL)], xsem).wait()

    plsc.subcore_barrier()

    @pl.when(jnp.logical_not(is_tail))
    def _():
        for j in range(NCH):
            pltpu.async_copy(x_v.at[pl.ds(j * CH, CH)],
                             acc_sh.at[ids_v.at[j]], sem, add=True)
        for j in range(NCH):
            pltpu.make_async_copy(x_v.at[pl.ds(j * CH, CH)],
                                  acc_sh.at[ids_v.at[j]], sem).wait()

    @pl.when(is_tail)
    def _():
        pltpu.async_copy(x_v.at[pl.ds(0, TAIL)],
                         acc_sh.at[ids_v.at[0]], sem, add=True)
        pltpu.make_async_copy(x_v.at[pl.ds(0, TAIL)],
                              acc_sh.at[ids_v.at[0]], sem).wait()

    plsc.subcore_barrier()

    @pl.when(sid == 0)
    def _():
        pltpu.sync_copy(acc_sh, ps_hbm.at[cid])

    pltpu.sync_copy(cnt_v, pc_hbm.at[wid])


@jax.jit
def _segment_partials(x, batch_i32, z128):
    mesh = plsc.VectorSubcoreMesh(core_axis_name="c", subcore_axis_name="s",
                                  num_cores=NC, num_subcores=NS)
    f = pl.kernel(
        _sc_body,
        out_type=(
            jax.ShapeDtypeStruct((NC, NUM_GRAPHS, D), jnp.float32),
            jax.ShapeDtypeStruct((NW, NUM_GRAPHS, 16), jnp.float32),
        ),
        mesh=mesh,
        scratch_types=[
            pltpu.VMEM((ROWS_PER, D), jnp.float32),
            pltpu.VMEM((NCH, CH), jnp.int32),
            pltpu.VMEM((NUM_GRAPHS, 16), jnp.float32),
            pltpu.VMEM_SHARED((NUM_GRAPHS, D), jnp.float32),
            pltpu.SemaphoreType.DMA,
            pltpu.SemaphoreType.DMA,
        ],
        compiler_params=pltpu.CompilerParams(use_tc_tiling_on_sc=True),
    )
    return f(x, batch_i32, z128)


def _tc_body(ps_ref, pc_ref, u_ref, w1_ref, b1_ref, w2_ref, b2_ref, y_ref):
    sums = ps_ref[0] + ps_ref[1]
    cnt = jnp.sum(pc_ref[...], axis=0)[:, :1]
    agg = sums / jnp.maximum(cnt, 1.0)
    out = jnp.concatenate([u_ref[...], agg], axis=1)
    h = jnp.maximum(out @ w1_ref[...] + b1_ref[...], 0.0)
    y_ref[...] = h @ w2_ref[...] + b2_ref[...]


@jax.jit
def _pool_mlp(ps, pc, u, W1, b1, W2, b2):
    return pl.pallas_call(
        _tc_body,
        out_shape=jax.ShapeDtypeStruct((NUM_GRAPHS, 32), jnp.float32),
    )(ps, pc, u, W1, b1.reshape(1, 32), W2, b2.reshape(1, 32))


def kernel(x, edge_index, edge_attr, u, batch, W1, b1, W2, b2):
    del edge_index, edge_attr
    batch_i32 = batch.astype(jnp.int32)
    z128 = jnp.zeros((NUM_GRAPHS, D), jnp.float32)
    ps, pc = _segment_partials(x, batch_i32, z128)
    return _pool_mlp(ps, pc, u, W1, b1, W2, b2)


# speedup vs baseline: 1.1400x; 1.0486x over previous
import jax
import jax.numpy as jnp
from jax import lax
from jax.experimental import pallas as pl
from jax.experimental.pallas import tpu as pltpu
from jax.experimental.pallas import tpu_sc as plsc

N_ROWS = 10000
D = 128
NUM_GRAPHS = 64
NC, NS = 2, 16
NW = NC * NS
ROWS_PER = 320
TAIL = N_ROWS - ROWS_PER * (NW - 1)
CH = 80
NCH = ROWS_PER // CH


def _sc_body(x_hbm, b_hbm, z128_hbm, ps_hbm, pc_hbm,
             x_v, ids_v, cnt_v, acc_sh, sem, xsem):
    cid = lax.axis_index("c")
    sid = lax.axis_index("s")
    wid = sid * NC + cid
    base = wid * ROWS_PER
    is_tail = wid == NW - 1

    @pl.when(sid == 0)
    def _():
        pltpu.sync_copy(z128_hbm, acc_sh)

    @pl.when(jnp.logical_not(is_tail))
    def _():
        for j in range(NCH):
            pltpu.sync_copy(b_hbm.at[pl.ds(base + j * CH, CH)], ids_v.at[j])
        pltpu.async_copy(x_hbm.at[pl.ds(base, ROWS_PER)], x_v, xsem)

    @pl.when(is_tail)
    def _():
        pltpu.sync_copy(b_hbm.at[pl.ds(base, TAIL)], ids_v.at[0])
        pltpu.async_copy(x_hbm.at[pl.ds(base, TAIL)],
                         x_v.at[pl.ds(0, TAIL)], xsem)

    zero16 = jnp.zeros((16,), jnp.float32)
    ones16 = jnp.ones((16,), jnp.float32)

    def zb(i, c):
        cnt_v[i] = zero16
        return c

    lax.fori_loop(0, NUM_GRAPHS, zb, 0)

    def grp(g, c):
        idv = ids_v[g // (CH // 16), pl.ds((g % (CH // 16)) * 16, 16)]
        for l in range(16):
            plsc.addupdate(cnt_v.at[idv[l]], ones16)
        return c

    ngrp = jnp.where(is_tail, TAIL // 16, ROWS_PER // 16)
    lax.fori_loop(0, ngrp, grp, 0)

    @pl.when(jnp.logical_not(is_tail))
    def _():
        pltpu.make_async_copy(x_hbm.at[pl.ds(base, ROWS_PER)], x_v, xsem).wait()

    @pl.when(is_tail)
    def _():
        pltpu.make_async_copy(x_hbm.at[pl.ds(base, TAIL)],
                              x_v.at[pl.ds(0, TAIL)], xsem).wait()

    plsc.subcore_barrier()

    @pl.when(jnp.logical_not(is_tail))
    def _():
        for j in range(NCH):
            pltpu.async_copy(x_v.at[pl.ds(j * CH, CH)],
                             acc_sh.at[ids_v.at[j]], sem, add=True)
        for j in range(NCH):
            pltpu.make_async_copy(x_v.at[pl.ds(j * CH, CH)],
                                  acc_sh.at[ids_v.at[j]], sem).wait()

    @pl.when(is_tail)
    def _():
        pltpu.async_copy(x_v.at[pl.ds(0, TAIL)],
                         acc_sh.at[ids_v.at[0]], sem, add=True)
        pltpu.make_async_copy(x_v.at[pl.ds(0, TAIL)],
                              acc_sh.at[ids_v.at[0]], sem).wait()

    plsc.subcore_barrier()

    @pl.when(sid == 0)
    def _():
        pltpu.sync_copy(acc_sh, ps_hbm.at[cid])

    pltpu.sync_copy(cnt_v, pc_hbm.at[wid])


@jax.jit
def _segment_partials(x, batch_i32, z128):
    mesh = plsc.VectorSubcoreMesh(core_axis_name="c", subcore_axis_name="s",
                                  num_cores=NC, num_subcores=NS)
    f = pl.kernel(
        _sc_body,
        out_type=(
            jax.ShapeDtypeStruct((NC, NUM_GRAPHS, D), jnp.float32),
            jax.ShapeDtypeStruct((NW, NUM_GRAPHS, 16), jnp.float32),
        ),
        mesh=mesh,
        scratch_types=[
            pltpu.VMEM((ROWS_PER, D), jnp.float32),
            pltpu.VMEM((NCH, CH), jnp.int32),
            pltpu.VMEM((NUM_GRAPHS, 16), jnp.float32),
            pltpu.VMEM_SHARED((NUM_GRAPHS, D), jnp.float32),
            pltpu.SemaphoreType.DMA,
            pltpu.SemaphoreType.DMA,
        ],
        compiler_params=pltpu.CompilerParams(use_tc_tiling_on_sc=True),
    )
    return f(x, batch_i32, z128)


def _tc_body(ps_ref, pc_ref, u_ref, w1t_ref, b1_ref, w2_ref, b2_ref, yt_ref):
    sums = ps_ref[0] + ps_ref[1]
    cnt = jnp.sum(pc_ref[...], axis=0)[:, :1]
    agg = sums / jnp.maximum(cnt, 1.0)
    out = jnp.concatenate([u_ref[...], agg], axis=1)
    h1 = lax.dot_general(out, w1t_ref[...],
                         (((1,), (1,)), ((), ())))
    h = jnp.maximum(h1 + b1_ref[...], 0.0)
    y = h @ w2_ref[...] + b2_ref[...]
    yt_ref[...] = y.T


@jax.jit
def _pool_mlp(ps, pc, u, W1, b1, W2, b2):
    yt = pl.pallas_call(
        _tc_body,
        out_shape=jax.ShapeDtypeStruct((32, NUM_GRAPHS), jnp.float32),
    )(ps, pc, u, W1.T, b1.reshape(1, 32), W2, b2.reshape(1, 32))
    return yt.T


def kernel(x, edge_index, edge_attr, u, batch, W1, b1, W2, b2):
    del edge_index, edge_attr
    batch_i32 = batch.astype(jnp.int32)
    z128 = jnp.zeros((NUM_GRAPHS, D), jnp.float32)
    ps, pc = _segment_partials(x, batch_i32, z128)
    return _pool_mlp(ps, pc, u, W1, b1, W2, b2)


# speedup vs baseline: 1.1879x; 1.0421x over previous
import jax
import jax.numpy as jnp
from jax import lax
from jax.experimental import pallas as pl
from jax.experimental.pallas import tpu as pltpu
from jax.experimental.pallas import tpu_sc as plsc

N_ROWS = 10000
D = 128
NUM_GRAPHS = 64
NC, NS = 2, 16
NW = NC * NS
ROWS_PER = 320
TAIL = N_ROWS - ROWS_PER * (NW - 1)
CH = 80
NCH = ROWS_PER // CH


def _sc_body(x_hbm, b_hbm, ps_hbm, pc_hbm,
             x_v, ids_v, cnt_v, zbuf_v, acc_sh, sem, xsem):
    cid = lax.axis_index("c")
    sid = lax.axis_index("s")
    wid = sid * NC + cid
    base = wid * ROWS_PER
    is_tail = wid == NW - 1

    zero16 = jnp.zeros((16,), jnp.float32)
    ones16 = jnp.ones((16,), jnp.float32)

    @pl.when(jnp.logical_not(is_tail))
    def _():
        for j in range(NCH):
            pltpu.sync_copy(b_hbm.at[pl.ds(base + j * CH, CH)], ids_v.at[j])
        for j in range(NCH):
            pltpu.async_copy(x_hbm.at[pl.ds(base + j * CH, CH)],
                             x_v.at[pl.ds(j * CH, CH)], xsem)

    @pl.when(is_tail)
    def _():
        pltpu.sync_copy(b_hbm.at[pl.ds(base, TAIL)], ids_v.at[0])
        pltpu.async_copy(x_hbm.at[pl.ds(base, TAIL)],
                         x_v.at[pl.ds(0, TAIL)], xsem)

    @pl.when(sid == 0)
    def _():
        def zrow(i, c):
            for k in range(D // 16):
                zbuf_v[i, pl.ds(k * 16, 16)] = zero16
            return c

        lax.fori_loop(0, NUM_GRAPHS, zrow, 0)
        pltpu.sync_copy(zbuf_v, acc_sh)

    def zb(i, c):
        cnt_v[i] = zero16
        return c

    lax.fori_loop(0, NUM_GRAPHS, zb, 0)

    def grp(g, c):
        idv = ids_v[g // (CH // 16), pl.ds((g % (CH // 16)) * 16, 16)]
        for l in range(16):
            plsc.addupdate(cnt_v.at[idv[l]], ones16)
        return c

    ngrp = jnp.where(is_tail, TAIL // 16, ROWS_PER // 16)
    lax.fori_loop(0, ngrp, grp, 0)

    plsc.subcore_barrier()

    @pl.when(jnp.logical_not(is_tail))
    def _():
        for j in range(NCH):
            pltpu.make_async_copy(x_hbm.at[pl.ds(base + j * CH, CH)],
                                  x_v.at[pl.ds(j * CH, CH)], xsem).wait()
        for j in range(NCH):
            pltpu.async_copy(x_v.at[pl.ds(j * CH, CH)],
                             acc_sh.at[ids_v.at[j]], sem, add=True)
        for j in range(NCH):
            pltpu.make_async_copy(x_v.at[pl.ds(j * CH, CH)],
                                  acc_sh.at[ids_v.at[j]], sem).wait()

    @pl.when(is_tail)
    def _():
        pltpu.make_async_copy(x_hbm.at[pl.ds(base, TAIL)],
                              x_v.at[pl.ds(0, TAIL)], xsem).wait()
        pltpu.async_copy(x_v.at[pl.ds(0, TAIL)],
                         acc_sh.at[ids_v.at[0]], sem, add=True)
        pltpu.make_async_copy(x_v.at[pl.ds(0, TAIL)],
                              acc_sh.at[ids_v.at[0]], sem).wait()

    plsc.subcore_barrier()

    @pl.when(sid == 0)
    def _():
        pltpu.sync_copy(acc_sh, ps_hbm.at[cid])

    pltpu.sync_copy(cnt_v, pc_hbm.at[wid])


@jax.jit
def _segment_partials(x, batch_i32):
    mesh = plsc.VectorSubcoreMesh(core_axis_name="c", subcore_axis_name="s",
                                  num_cores=NC, num_subcores=NS)
    f = pl.kernel(
        _sc_body,
        out_type=(
            jax.ShapeDtypeStruct((NC, NUM_GRAPHS, D), jnp.float32),
            jax.ShapeDtypeStruct((NW, NUM_GRAPHS, 16), jnp.float32),
        ),
        mesh=mesh,
        scratch_types=[
            pltpu.VMEM((ROWS_PER, D), jnp.float32),
            pltpu.VMEM((NCH, CH), jnp.int32),
            pltpu.VMEM((NUM_GRAPHS, 16), jnp.float32),
            pltpu.VMEM((NUM_GRAPHS, D), jnp.float32),
            pltpu.VMEM_SHARED((NUM_GRAPHS, D), jnp.float32),
            pltpu.SemaphoreType.DMA,
            pltpu.SemaphoreType.DMA,
        ],
        compiler_params=pltpu.CompilerParams(use_tc_tiling_on_sc=True),
    )
    return f(x, batch_i32)


def _tc_body(ps_ref, pc_ref, u_ref, w1t_ref, b1_ref, w2_ref, b2_ref, yt_ref):
    sums = ps_ref[0] + ps_ref[1]
    cnt = jnp.sum(pc_ref[...], axis=0)[:, :1]
    agg = sums / jnp.maximum(cnt, 1.0)
    out = jnp.concatenate([u_ref[...], agg], axis=1)
    h1 = lax.dot_general(out, w1t_ref[...],
                         (((1,), (1,)), ((), ())))
    h = jnp.maximum(h1 + b1_ref[...], 0.0)
    y = h @ w2_ref[...] + b2_ref[...]
    yt_ref[...] = y.T


@jax.jit
def _pool_mlp(ps, pc, u, W1, b1, W2, b2):
    yt = pl.pallas_call(
        _tc_body,
        out_shape=jax.ShapeDtypeStruct((32, NUM_GRAPHS), jnp.float32),
    )(ps, pc, u, W1.T, b1.reshape(1, 32), W2, b2.reshape(1, 32))
    return yt.T


def kernel(x, edge_index, edge_attr, u, batch, W1, b1, W2, b2):
    del edge_index, edge_attr
    batch_i32 = batch.astype(jnp.int32)
    ps, pc = _segment_partials(x, batch_i32)
    return _pool_mlp(ps, pc, u, W1, b1, W2, b2)
